# Initial kernel scaffold; baseline (speedup 1.0000x reference)
import jax
import jax.numpy as jnp
from jax.experimental import pallas as pl


def kernel(x, Wr, br, W1, b1, W2, b2):
    raise NotImplementedError("write your pallas kernel here")


# speedup vs baseline: 1.2659x; 1.2659x over previous
import functools

import jax
import jax.numpy as jnp
from jax.experimental import pallas as pl
from jax.experimental.pallas import tpu as pltpu

T, D, E, H, TOP_K = 2048, 1024, 8, 2048, 2
HB = 512
NEG_INF = float("-inf")


def _router_body(x_ref, wr_ref, br_ref, sel_ref, gates_ref, combine_ref,
                 ent_ref, lb_ref):
    x = x_ref[...]
    wr = wr_ref[...]
    logits = jnp.dot(x, wr, preferred_element_type=jnp.float32)
    logits = logits + br_ref[...][None, :]

    iota = jax.lax.broadcasted_iota(jnp.int32, (T, E), 1)
    v1 = jnp.max(logits, axis=1, keepdims=True)
    i1 = jnp.min(jnp.where(logits == v1, iota, E), axis=1, keepdims=True)
    l2 = jnp.where(iota == i1, NEG_INF, logits)
    v2 = jnp.max(l2, axis=1, keepdims=True)
    i2 = jnp.min(jnp.where(l2 == v2, iota, E), axis=1, keepdims=True)

    sel_ref[:, 0:1] = i1
    sel_ref[:, 1:2] = i2

    e2 = jnp.exp(v2 - v1)
    denom = 1.0 + e2
    g1 = 1.0 / denom
    g2 = e2 / denom
    gates_ref[:, 0:1] = g1
    gates_ref[:, 1:2] = g2
    combine_ref[...] = (jnp.where(iota == i1, g1, 0.0)
                        + jnp.where(iota == i2, g2, 0.0))

    ex = jnp.exp(logits - v1)
    p = ex / jnp.sum(ex, axis=1, keepdims=True)
    ent = -(p * jnp.log(jnp.clip(p, 1e-8, None)))
    ent_ref[...] = jnp.sum(ent, axis=(0, 1), keepdims=True) / T
    imp = jnp.mean(p, axis=0, keepdims=True)
    lb_ref[...] = jnp.sum((imp - 1.0 / E) ** 2, axis=(0, 1),
                          keepdims=True) / E


def _router(x, Wr, br):
    return pl.pallas_call(
        _router_body,
        out_shape=(
            jax.ShapeDtypeStruct((T, TOP_K), jnp.int32),
            jax.ShapeDtypeStruct((T, TOP_K), jnp.float32),
            jax.ShapeDtypeStruct((T, E), jnp.float32),
            jax.ShapeDtypeStruct((1, 1), jnp.float32),
            jax.ShapeDtypeStruct((1, 1), jnp.float32),
        ),
    )(x, Wr, br)


def _ffn_body(x_ref, w1_ref, b1_ref, w2_ref, b2_ref, comb_ref, out_ref,
              acc_ref):
    e = pl.program_id(0)
    hc = pl.program_id(1)
    x = x_ref[...]
    h = jnp.dot(x, w1_ref[0], preferred_element_type=jnp.float32)
    h = jnp.maximum(h + b1_ref[0], 0.0)
    part = jnp.dot(h, w2_ref[0], preferred_element_type=jnp.float32)
    iota = jax.lax.broadcasted_iota(jnp.int32, (T, E), 1)
    comb = jnp.sum(jnp.where(iota == e, comb_ref[...], 0.0),
                   axis=1, keepdims=True)
    contrib = part * comb

    @pl.when(jnp.logical_and(e == 0, hc == 0))
    def _init():
        acc_ref[...] = jnp.zeros_like(acc_ref)

    @pl.when(hc == 0)
    def _bias():
        acc_ref[...] += b2_ref[0] * comb

    acc_ref[...] += contrib

    @pl.when(jnp.logical_and(e == E - 1, hc == H // HB - 1))
    def _done():
        out_ref[...] = acc_ref[...]


def _ffn(x, W1, b1, W2, b2, combine):
    nh = H // HB
    grid = (E, nh)
    return pl.pallas_call(
        _ffn_body,
        grid=grid,
        in_specs=[
            pl.BlockSpec((T, D), lambda e, hc: (0, 0)),
            pl.BlockSpec((1, D, HB), lambda e, hc: (e, 0, hc)),
            pl.BlockSpec((1, 1, HB), lambda e, hc: (e, 0, hc)),
            pl.BlockSpec((1, HB, D), lambda e, hc: (e, hc, 0)),
            pl.BlockSpec((1, 1, D), lambda e, hc: (e, 0, 0)),
            pl.BlockSpec((T, E), lambda e, hc: (0, 0)),
        ],
        out_specs=pl.BlockSpec((T, D), lambda e, hc: (0, 0)),
        out_shape=jax.ShapeDtypeStruct((T, D), jnp.float32),
        scratch_shapes=[pltpu.VMEM((T, D), jnp.float32)],
    )(x, W1, b1.reshape(E, 1, H), W2, b2.reshape(E, 1, D), combine)


def kernel(x, Wr, br, W1, b1, W2, b2):
    sel, gates, combine, ent, lb = _router(x, Wr, br)
    mixed = _ffn(x, W1, b1, W2, b2, combine)
    return (mixed, ent.reshape(()), lb.reshape(()), sel)


# speedup vs baseline: 1.5581x; 1.2308x over previous
import functools

import jax
import jax.numpy as jnp
from jax import lax
from jax.experimental import pallas as pl
from jax.experimental.pallas import tpu as pltpu
from jax.experimental.pallas import tpu_sc as plsc

T, D, E, H, TOP_K = 2048, 1024, 8, 2048, 2
TCHUNK = 512
NCHUNK = T // TCHUNK
B = 256
NBLK = T * TOP_K // B + E - 1
CAP = NBLK * B
NW = 32
TPW = T // NW
NEG_INF = float("-inf")


def _router_body(x_ref, wr_ref, br_ref,
                 sel_ref, gates_ref, r0_ref, r1_ref, gs0_ref, gs1_ref,
                 cnt_ref, ent_ref, lb_ref,
                 carry, impacc, entacc):
    c = pl.program_id(0)
    logits = jnp.dot(x_ref[...], wr_ref[...],
                     preferred_element_type=jnp.float32)
    logits = logits + br_ref[...]

    iota = jax.lax.broadcasted_iota(jnp.int32, (TCHUNK, E), 1)
    v1 = jnp.max(logits, axis=1, keepdims=True)
    i1 = jnp.min(jnp.where(logits == v1, iota, E), axis=1, keepdims=True)
    l2 = jnp.where(iota == i1, NEG_INF, logits)
    v2 = jnp.max(l2, axis=1, keepdims=True)
    i2 = jnp.min(jnp.where(l2 == v2, iota, E), axis=1, keepdims=True)
    sel_ref[:, 0:1] = i1
    sel_ref[:, 1:2] = i2

    e2 = jnp.exp(v2 - v1)
    g1 = 1.0 / (1.0 + e2)
    g2 = e2 / (1.0 + e2)
    gates_ref[:, 0:1] = g1
    gates_ref[:, 1:2] = g2

    o0 = (iota == i1).astype(jnp.float32)
    o1 = (iota == i2).astype(jnp.float32)

    gs0_ref[...] = jnp.broadcast_to(g1, (TCHUNK, 128))
    gs1_ref[...] = jnp.broadcast_to(g2, (TCHUNK, 128))

    cvec = o0 + o1
    ir = jax.lax.broadcasted_iota(jnp.int32, (TCHUNK, TCHUNK), 0)
    ic = jax.lax.broadcasted_iota(jnp.int32, (TCHUNK, TCHUNK), 1)
    lex = (ic < ir).astype(jnp.float32)

    @pl.when(c == 0)
    def _init():
        carry[...] = jnp.zeros_like(carry)
        impacc[...] = jnp.zeros_like(impacc)
        entacc[...] = jnp.zeros_like(entacc)

    cex = jnp.dot(lex, cvec, preferred_element_type=jnp.float32,
                  precision=jax.lax.Precision.HIGHEST) + carry[...]
    r0_ref[...] = jnp.sum(o0 * cex, axis=1, keepdims=True)
    r1_ref[...] = jnp.sum(o1 * (cex + o0), axis=1, keepdims=True)
    carry[...] += jnp.sum(cvec, axis=0, keepdims=True)

    ex = jnp.exp(logits - v1)
    p = ex / jnp.sum(ex, axis=1, keepdims=True)
    ent = -(p * jnp.log(jnp.clip(p, 1e-8, None)))
    entacc[...] += jnp.sum(ent, axis=(0, 1), keepdims=True)
    impacc[...] += jnp.sum(p, axis=0, keepdims=True)

    @pl.when(c == NCHUNK - 1)
    def _finalize():
        cnt_ref[...] = carry[...].astype(jnp.int32)
        ent_ref[...] = entacc[...] / T
        imp = impacc[...] / T
        lb_ref[...] = jnp.sum((imp - 1.0 / E) ** 2, axis=(0, 1),
                              keepdims=True) / E


def _router(x, Wr, br):
    return pl.pallas_call(
        _router_body,
        grid=(NCHUNK,),
        in_specs=[
            pl.BlockSpec((TCHUNK, D), lambda c: (c, 0)),
            pl.BlockSpec((D, E), lambda c: (0, 0)),
            pl.BlockSpec((1, E), lambda c: (0, 0)),
        ],
        out_specs=(
            pl.BlockSpec((TCHUNK, TOP_K), lambda c: (c, 0)),
            pl.BlockSpec((TCHUNK, TOP_K), lambda c: (c, 0)),
            pl.BlockSpec((TCHUNK, 1), lambda c: (c, 0)),
            pl.BlockSpec((TCHUNK, 1), lambda c: (c, 0)),
            pl.BlockSpec((TCHUNK, 128), lambda c: (c, 0)),
            pl.BlockSpec((TCHUNK, 128), lambda c: (c, 0)),
            pl.BlockSpec((1, E), lambda c: (0, 0)),
            pl.BlockSpec((1, 1), lambda c: (0, 0)),
            pl.BlockSpec((1, 1), lambda c: (0, 0)),
        ),
        out_shape=(
            jax.ShapeDtypeStruct((T, TOP_K), jnp.int32),
            jax.ShapeDtypeStruct((T, TOP_K), jnp.float32),
            jax.ShapeDtypeStruct((T, 1), jnp.float32),
            jax.ShapeDtypeStruct((T, 1), jnp.float32),
            jax.ShapeDtypeStruct((T, 128), jnp.float32),
            jax.ShapeDtypeStruct((T, 128), jnp.float32),
            jax.ShapeDtypeStruct((1, E), jnp.int32),
            jax.ShapeDtypeStruct((1, 1), jnp.float32),
            jax.ShapeDtypeStruct((1, 1), jnp.float32),
        ),
        scratch_shapes=[
            pltpu.VMEM((1, E), jnp.float32),
            pltpu.VMEM((1, E), jnp.float32),
            pltpu.VMEM((1, 1), jnp.float32),
        ],
    )(x, Wr, br.reshape(1, E))


def _pos_body(sel_ref, r0_ref, r1_ref, cnt_ref, pos0_ref, pos1_ref):
    cnt = cnt_ref[...].astype(jnp.float32)
    padded = jnp.floor((cnt + (B - 1)) * (1.0 / B)) * B
    jota = jax.lax.broadcasted_iota(jnp.int32, (T, E), 1).astype(jnp.float32)
    e1f = sel_ref[:, 0:1].astype(jnp.float32)
    e2f = sel_ref[:, 1:2].astype(jnp.float32)
    off0 = jnp.sum(jnp.where(jota < e1f, padded, 0.0), axis=1, keepdims=True)
    off1 = jnp.sum(jnp.where(jota < e2f, padded, 0.0), axis=1, keepdims=True)
    pos0_ref[...] = (r0_ref[...] + off0).astype(jnp.int32)
    pos1_ref[...] = (r1_ref[...] + off1).astype(jnp.int32)


def _pos(sel, r0, r1, cnt):
    return pl.pallas_call(
        _pos_body,
        out_shape=(
            jax.ShapeDtypeStruct((T, 1), jnp.int32),
            jax.ShapeDtypeStruct((T, 1), jnp.int32),
        ),
    )(sel, r0, r1, cnt)


def _plan_body(cnt_ref, be_ref, nv_ref):
    offs = []
    idx = jnp.int32(0)
    for e in range(E):
        offs.append(idx)
        idx = idx + (cnt_ref[e] + (B - 1)) // B
    for j in range(NBLK):
        be = jnp.int32(-1)
        for e in range(E):
            be = be + jnp.where(j >= offs[e], 1, 0).astype(jnp.int32)
        be_ref[j] = jnp.maximum(be, 0)
    nv_ref[0] = idx


def _plan(cnt):
    return pl.pallas_call(
        _plan_body,
        in_specs=[pl.BlockSpec(memory_space=pltpu.SMEM)],
        out_specs=(
            pl.BlockSpec(memory_space=pltpu.SMEM),
            pl.BlockSpec(memory_space=pltpu.SMEM),
        ),
        out_shape=(
            jax.ShapeDtypeStruct((NBLK,), jnp.int32),
            jax.ShapeDtypeStruct((1,), jnp.int32),
        ),
    )(cnt)


def _dispatch_body(x_hbm, pos0_hbm, pos1_hbm, gs0_hbm, gs1_hbm,
                   xs_hbm, gsrt_hbm,
                   idx0_v, idx1_v, x_v, g0_v, g1_v, sem):
    wid = lax.axis_index("s") * 2 + lax.axis_index("c")
    base = wid * TPW
    pltpu.sync_copy(pos0_hbm.at[pl.ds(base, TPW)], idx0_v)
    pltpu.sync_copy(pos1_hbm.at[pl.ds(base, TPW)], idx1_v)
    pltpu.sync_copy(x_hbm.at[pl.ds(base, TPW)], x_v)
    pltpu.sync_copy(gs0_hbm.at[pl.ds(base, TPW)], g0_v)
    pltpu.sync_copy(gs1_hbm.at[pl.ds(base, TPW)], g1_v)
    c0 = pltpu.async_copy(x_v, xs_hbm.at[idx0_v], sem)
    c1 = pltpu.async_copy(x_v, xs_hbm.at[idx1_v], sem)
    c2 = pltpu.async_copy(g0_v, gsrt_hbm.at[idx0_v], sem)
    c3 = pltpu.async_copy(g1_v, gsrt_hbm.at[idx1_v], sem)
    c0.wait()
    c1.wait()
    c2.wait()
    c3.wait()


def _dispatch(x, pos0, pos1, gs0, gs1):
    mesh = plsc.VectorSubcoreMesh(core_axis_name="c", subcore_axis_name="s")
    k = pl.kernel(
        _dispatch_body,
        out_type=(
            jax.ShapeDtypeStruct((CAP, D), jnp.float32),
            jax.ShapeDtypeStruct((CAP, 128), jnp.float32),
        ),
        mesh=mesh,
        scratch_types=[
            pltpu.VMEM((TPW,), jnp.int32),
            pltpu.VMEM((TPW,), jnp.int32),
            pltpu.VMEM((TPW, D), jnp.float32),
            pltpu.VMEM((TPW, 128), jnp.float32),
            pltpu.VMEM((TPW, 128), jnp.float32),
            pltpu.SemaphoreType.DMA,
        ],
    )
    return k(x, pos0, pos1, gs0, gs1)


def _ffn_body(be_ref, nv_ref, xs_ref, w1_ref, b1_ref, w2_ref, b2_ref,
              gs_ref, ys_ref):
    b = pl.program_id(0)

    @pl.when(b < nv_ref[0])
    def _compute():
        h = jnp.dot(xs_ref[...], w1_ref[0],
                    preferred_element_type=jnp.float32)
        h = jnp.maximum(h + b1_ref[0], 0.0)
        o = jnp.dot(h, w2_ref[0], preferred_element_type=jnp.float32)
        o = o + b2_ref[0]
        ys_ref[...] = o * gs_ref[:, 0:1]


def _ffn(xs, W1, b1, W2, b2, gsrt, be, nv):
    grid_spec = pltpu.PrefetchScalarGridSpec(
        num_scalar_prefetch=2,
        grid=(NBLK,),
        in_specs=[
            pl.BlockSpec((B, D), lambda b, be, nv: (b, 0)),
            pl.BlockSpec((1, D, H), lambda b, be, nv: (be[b], 0, 0)),
            pl.BlockSpec((1, 1, H), lambda b, be, nv: (be[b], 0, 0)),
            pl.BlockSpec((1, H, D), lambda b, be, nv: (be[b], 0, 0)),
            pl.BlockSpec((1, 1, D), lambda b, be, nv: (be[b], 0, 0)),
            pl.BlockSpec((B, 128), lambda b, be, nv: (b, 0)),
        ],
        out_specs=pl.BlockSpec((B, D), lambda b, be, nv: (b, 0)),
    )
    return pl.pallas_call(
        _ffn_body,
        grid_spec=grid_spec,
        out_shape=jax.ShapeDtypeStruct((CAP, D), jnp.float32),
    )(be, nv, xs, W1, b1.reshape(E, 1, H), W2, b2.reshape(E, 1, D), gsrt)


def _combine_body(ys_hbm, pos0_hbm, pos1_hbm, out_hbm,
                  idx0_v, idx1_v, a_v, b_v, sem):
    wid = lax.axis_index("s") * 2 + lax.axis_index("c")
    for half in range(2):
        base = wid * TPW + half * (TPW // 2)
        pltpu.sync_copy(pos0_hbm.at[pl.ds(base, TPW // 2)], idx0_v)
        pltpu.sync_copy(pos1_hbm.at[pl.ds(base, TPW // 2)], idx1_v)
        ca = pltpu.async_copy(ys_hbm.at[idx0_v], a_v, sem)
        cb = pltpu.async_copy(ys_hbm.at[idx1_v], b_v, sem)
        ca.wait()
        cb.wait()

        def _row(r, _):
            for j in range(D // 16):
                sl = pl.ds(j * 16, 16)
                a_v[r, sl] = a_v[r, sl] + b_v[r, sl]
            return 0

        lax.fori_loop(0, TPW // 2, _row, 0)
        pltpu.sync_copy(a_v, out_hbm.at[pl.ds(base, TPW // 2)])


def _combine(ys, pos0, pos1):
    mesh = plsc.VectorSubcoreMesh(core_axis_name="c", subcore_axis_name="s")
    k = pl.kernel(
        _combine_body,
        out_type=jax.ShapeDtypeStruct((T, D), jnp.float32),
        mesh=mesh,
        scratch_types=[
            pltpu.VMEM((TPW // 2,), jnp.int32),
            pltpu.VMEM((TPW // 2,), jnp.int32),
            pltpu.VMEM((TPW // 2, D), jnp.float32),
            pltpu.VMEM((TPW // 2, D), jnp.float32),
            pltpu.SemaphoreType.DMA,
        ],
    )
    return k(ys, pos0, pos1)


def kernel(x, Wr, br, W1, b1, W2, b2):
    (sel, _gates, r0, r1, gs0, gs1, cnt, ent, lb) = _router(x, Wr, br)
    pos0, pos1 = _pos(sel, r0, r1, cnt)
    be, nv = _plan(cnt.reshape(E))
    p0 = pos0.reshape(T)
    p1 = pos1.reshape(T)
    xs, gsrt = _dispatch(x, p0, p1, gs0, gs1)
    ys = _ffn(xs, W1, b1, W2, b2, gsrt, be, nv)
    mixed = _combine(ys, p0, p1)
    return (mixed, ent.reshape(()), lb.reshape(()), sel)
